# baseline (speedup 1.0000x reference)
import functools

import jax
import jax.numpy as jnp
from jax import lax
from jax.experimental import pallas as pl
from jax.experimental.pallas import tpu as pltpu
from jax.experimental.pallas import tpu_sc as plsc

VOCAB = 1000000
SEQ_LEN = 200
EMBED_DIM = 64
BATCH = 1024

NC, NS, L = 2, 16, 16
NW = NC * NS
CH = 100
NCHUNK = BATCH * SEQ_LEN // CH
CPW = NCHUNK // NW
SPC = SEQ_LEN // CH
SCALE = 8.0


def _body(x_hbm, table_hbm, pos_hbm, out_hbm, idx_v, gath_v, pos_v, sem):
    wid = lax.axis_index("s") * NC + lax.axis_index("c")
    c0 = wid * CPW

    pltpu.sync_copy(pos_hbm, pos_v)

    def chunk_body(j, carry):
        c = c0 + j
        pltpu.sync_copy(x_hbm.at[c], idx_v)
        pltpu.async_copy(table_hbm.at[idx_v], gath_v, sem).wait()
        p = lax.rem(c, SPC)

        def row_body(r, carry2):
            for g in range(EMBED_DIM // L):
                sl = pl.ds(g * L, L)
                gath_v[r, sl] = gath_v[r, sl] * SCALE + pos_v[p, r, sl]
            return carry2

        lax.fori_loop(0, CH, row_body, 0, unroll=2)
        pltpu.sync_copy(gath_v, out_hbm.at[c])
        return carry

    lax.fori_loop(0, CPW, chunk_body, 0)


@functools.partial(jax.jit, static_argnames=())
def kernel(x, token_table, pos_table):
    x2 = x.reshape(NCHUNK, CH).astype(jnp.int32)
    pos3 = pos_table.reshape(SPC, CH, EMBED_DIM)
    mesh = plsc.VectorSubcoreMesh(
        core_axis_name="c", subcore_axis_name="s",
        num_cores=NC, num_subcores=NS)
    out = pl.kernel(
        _body,
        out_type=jax.ShapeDtypeStruct((NCHUNK, CH, EMBED_DIM), jnp.float32),
        mesh=mesh,
        scratch_types=[
            pltpu.VMEM((CH,), jnp.int32),
            pltpu.VMEM((CH, EMBED_DIM), jnp.float32),
            pltpu.VMEM((SPC, CH, EMBED_DIM), jnp.float32),
            pltpu.SemaphoreType.DMA,
        ],
        compiler_params=pltpu.CompilerParams(use_tc_tiling_on_sc=False),
    )(x2, token_table, pos3)
    return out.reshape(BATCH, SEQ_LEN, EMBED_DIM)


# speedup vs baseline: 1.0847x; 1.0847x over previous
import jax
import jax.numpy as jnp
from jax import lax
from jax.experimental import pallas as pl
from jax.experimental.pallas import tpu as pltpu
from jax.experimental.pallas import tpu_sc as plsc

VOCAB = 1000000
SEQ_LEN = 200
EMBED_DIM = 64
BATCH = 1024

NC, NS, L = 2, 16, 16
NW = NC * NS
IPW = BATCH // NW
IPC = 2
RPC = IPC * SEQ_LEN
NCH = IPW // IPC
GSZ = 100
NG = RPC // GSZ
SPI = SEQ_LEN // GSZ
SCALE = 8.0


def _body(x_hbm, table_hbm, pos8_hbm, out_hbm,
          i0, i1, g0, g1,
          sg0, sg1, so0, so1, sp0, sp1):
    wid = lax.axis_index("s") * NC + lax.axis_index("c")
    c_base = wid * IPW * SPI

    ibufs = (i0, i1)
    gbufs = (g0, g1)
    sgs = (sg0, sg1)
    sos = (so0, so1)
    sps = (sp0, sp1)

    def issue(j, sl, first=False):
        ibuf, g, sg, so, sp = ibufs[sl], gbufs[sl], sgs[sl], sos[sl], sps[sl]
        if not first:
            pltpu.make_async_copy(g, out_hbm.at[pl.ds(0, NG)], so).wait()
        for i in range(IPC):
            pltpu.async_copy(pos8_hbm, g.at[pl.ds(i * SPI, SPI)], sp)
        c0 = c_base + j * NG
        pltpu.sync_copy(x_hbm.at[pl.ds(c0, NG)], ibuf)
        for i in range(IPC):
            pltpu.make_async_copy(
                pos8_hbm, g.at[pl.ds(i * SPI, SPI)], sp).wait()
        for k in range(NG):
            pltpu.async_copy(
                table_hbm.at[ibuf.at[k]], g.at[k], sg, add=True)

    def consume(j, sl):
        ibuf, g, sg, so = ibufs[sl], gbufs[sl], sgs[sl], sos[sl]
        for k in range(NG):
            pltpu.make_async_copy(
                table_hbm.at[ibuf.at[k]], g.at[k], sg).wait()
        for k in range(NG):
            def row_body(r, carry, _k=k):
                for q in range(EMBED_DIM // L):
                    qs = pl.ds(q * L, L)
                    g[_k, r, qs] = g[_k, r, qs] * SCALE
                return carry
            lax.fori_loop(0, GSZ, row_body, 0, unroll=4)
        c0 = c_base + j * NG
        pltpu.async_copy(g, out_hbm.at[pl.ds(c0, NG)], so)

    issue(0, 0, first=True)

    def pair_body(k, carry):
        @pl.when(k == 0)
        def _():
            issue(1, 1, first=True)

        @pl.when(k > 0)
        def _():
            issue(2 * k + 1, 1)

        consume(2 * k, 0)

        @pl.when(k < NCH // 2 - 1)
        def _():
            issue(2 * k + 2, 0)

        consume(2 * k + 1, 1)
        return carry

    lax.fori_loop(0, NCH // 2, pair_body, 0)

    for sl in range(2):
        pltpu.make_async_copy(
            gbufs[sl], out_hbm.at[pl.ds(0, NG)], sos[sl]).wait()


@jax.jit
def kernel(x, token_table, pos_table):
    x2 = x.reshape(BATCH * SEQ_LEN // GSZ, GSZ)
    pos8 = (pos_table * (1.0 / SCALE)).reshape(SPI, GSZ, EMBED_DIM)
    mesh = plsc.VectorSubcoreMesh(
        core_axis_name="c", subcore_axis_name="s",
        num_cores=NC, num_subcores=NS)
    out = pl.kernel(
        _body,
        out_type=jax.ShapeDtypeStruct(
            (BATCH * SEQ_LEN // GSZ, GSZ, EMBED_DIM), jnp.float32),
        mesh=mesh,
        scratch_types=[
            pltpu.VMEM((NG, GSZ), jnp.int32),
            pltpu.VMEM((NG, GSZ), jnp.int32),
            pltpu.VMEM((NG, GSZ, EMBED_DIM), jnp.float32),
            pltpu.VMEM((NG, GSZ, EMBED_DIM), jnp.float32),
            pltpu.SemaphoreType.DMA,
            pltpu.SemaphoreType.DMA,
            pltpu.SemaphoreType.DMA,
            pltpu.SemaphoreType.DMA,
            pltpu.SemaphoreType.DMA,
            pltpu.SemaphoreType.DMA,
        ],
        compiler_params=pltpu.CompilerParams(use_tc_tiling_on_sc=False),
    )(x2, token_table, pos8)
    return out.reshape(BATCH, SEQ_LEN, EMBED_DIM)


# speedup vs baseline: 1.5359x; 1.4160x over previous
import jax
import jax.numpy as jnp
from jax import lax
from jax.experimental import pallas as pl
from jax.experimental.pallas import tpu as pltpu
from jax.experimental.pallas import tpu_sc as plsc
from jax.experimental import layout as jex_layout

VOCAB = 1000000
SEQ_LEN = 200
EMBED_DIM = 64
BATCH = 1024

NC, NS, L = 2, 16, 16
NW = NC * NS
IPW = BATCH // NW
IPC = 2
RPC = IPC * SEQ_LEN
NCH = IPW // IPC
GSZ = 100
NG = RPC // GSZ
SPI = SEQ_LEN // GSZ
SCALE = 8.0


def _body(x_hbm, table_hbm, pos8_hbm, out_hbm,
          i0, i1, g0, g1,
          sg0, sg1, so0, so1, sp0, sp1):
    wid = lax.axis_index("s") * NC + lax.axis_index("c")
    c_base = wid * IPW * SPI

    ibufs = (i0, i1)
    gbufs = (g0, g1)
    sgs = (sg0, sg1)
    sos = (so0, so1)
    sps = (sp0, sp1)

    def issue(j, sl, first=False):
        ibuf, g, sg, so, sp = ibufs[sl], gbufs[sl], sgs[sl], sos[sl], sps[sl]
        if not first:
            pltpu.make_async_copy(g, out_hbm.at[pl.ds(0, NG)], so).wait()
        for i in range(IPC):
            pltpu.async_copy(pos8_hbm, g.at[pl.ds(i * SPI, SPI)], sp)
        c0 = c_base + j * NG
        pltpu.sync_copy(x_hbm.at[pl.ds(c0, NG)], ibuf)
        for i in range(IPC):
            pltpu.make_async_copy(
                pos8_hbm, g.at[pl.ds(i * SPI, SPI)], sp).wait()
        for k in range(NG):
            pltpu.async_copy(
                table_hbm.at[ibuf.at[k]], g.at[k], sg, add=True)

    def consume(j, sl):
        ibuf, g, sg, so = ibufs[sl], gbufs[sl], sgs[sl], sos[sl]
        for k in range(NG):
            pltpu.make_async_copy(
                table_hbm.at[ibuf.at[k]], g.at[k], sg).wait()
        for k in range(NG):
            def row_body(r, carry, _k=k):
                for q in range(EMBED_DIM // L):
                    qs = pl.ds(q * L, L)
                    g[_k, r, qs] = g[_k, r, qs] * SCALE
                return carry
            lax.fori_loop(0, GSZ, row_body, 0, unroll=4)
        c0 = c_base + j * NG
        pltpu.async_copy(g, out_hbm.at[pl.ds(c0, NG)], so)

    issue(0, 0, first=True)

    def pair_body(k, carry):
        @pl.when(k == 0)
        def _():
            issue(1, 1, first=True)

        @pl.when(k > 0)
        def _():
            issue(2 * k + 1, 1)

        consume(2 * k, 0)

        @pl.when(k < NCH // 2 - 1)
        def _():
            issue(2 * k + 2, 0)

        consume(2 * k + 1, 1)
        return carry

    lax.fori_loop(0, NCH // 2, pair_body, 0)

    for sl in range(2):
        pltpu.make_async_copy(
            gbufs[sl], out_hbm.at[pl.ds(0, NG)], sos[sl]).wait()


@jax.jit
def kernel(x, token_table, pos_table):
    token_table = jex_layout.with_layout_constraint(
        token_table,
        jex_layout.Layout(major_to_minor=(0, 1), tiling=((8,),)))
    x2 = x.reshape(BATCH * SEQ_LEN // GSZ, GSZ)
    pos8 = (pos_table * (1.0 / SCALE)).reshape(SPI, GSZ, EMBED_DIM)
    mesh = plsc.VectorSubcoreMesh(
        core_axis_name="c", subcore_axis_name="s",
        num_cores=NC, num_subcores=NS)
    out = pl.kernel(
        _body,
        out_type=jax.ShapeDtypeStruct(
            (BATCH * SEQ_LEN // GSZ, GSZ, EMBED_DIM), jnp.float32),
        mesh=mesh,
        scratch_types=[
            pltpu.VMEM((NG, GSZ), jnp.int32),
            pltpu.VMEM((NG, GSZ), jnp.int32),
            pltpu.VMEM((NG, GSZ, EMBED_DIM), jnp.float32),
            pltpu.VMEM((NG, GSZ, EMBED_DIM), jnp.float32),
            pltpu.SemaphoreType.DMA,
            pltpu.SemaphoreType.DMA,
            pltpu.SemaphoreType.DMA,
            pltpu.SemaphoreType.DMA,
            pltpu.SemaphoreType.DMA,
            pltpu.SemaphoreType.DMA,
        ],
        compiler_params=pltpu.CompilerParams(use_tc_tiling_on_sc=False),
    )(x2, token_table, pos8)
    return out.reshape(BATCH, SEQ_LEN, EMBED_DIM)
